# baseline (speedup 1.0000x reference)
import jax
import jax.numpy as jnp
from jax import lax
from jax.experimental import pallas as pl
from jax.experimental.pallas import tpu as pltpu
from jax.experimental.pallas import tpu_sc as plsc

_R, _N = 128, 32768
_HALF = _N // 2
_CHUNK = 2048
_NCHUNKS = _HALF // _CHUNK
_BIG = 2**30


def _sc_argmax_body(probs_hbm, maxes_hbm, idxs_hbm, buf_v, res_v, tmp_v, sem0, sem1):
    c = lax.axis_index("c")
    s = lax.axis_index("s")
    wid = c * 16 + s
    g = wid // 2
    h = wid % 2
    row0 = g * 8
    col0 = h * _HALF
    lanes = lax.iota(jnp.int32, 16)
    sems = [sem0, sem1]

    def chunk_src(ci):
        return probs_hbm.at[pl.ds(row0, 8), pl.ds(col0 + ci * _CHUNK, _CHUNK)]

    cp = pltpu.async_copy(chunk_src(0), buf_v.at[0], sems[0])
    maxv = [jnp.full((16,), -1.0, jnp.float32) for _ in range(8)]
    maxj = [jnp.zeros((16,), jnp.int32) for _ in range(8)]
    for ci in range(_NCHUNKS):
        b = ci % 2
        cp_next = None
        if ci + 1 < _NCHUNKS:
            cp_next = pltpu.async_copy(
                chunk_src(ci + 1), buf_v.at[(ci + 1) % 2], sems[(ci + 1) % 2]
            )
        cp.wait()

        def body(j, state):
            out = []
            jg = ci * (_CHUNK // 16) + j
            for r in range(8):
                m, mj = state[2 * r], state[2 * r + 1]
                x = buf_v[b, r, pl.ds(j * 16, 16)]
                gt = x > m
                out.append(jnp.where(gt, x, m))
                out.append(jnp.where(gt, jg, mj))
            return tuple(out)

        state = []
        for r in range(8):
            state += [maxv[r], maxj[r]]
        state = lax.fori_loop(0, _CHUNK // 16, body, tuple(state))
        for r in range(8):
            maxv[r], maxj[r] = state[2 * r], state[2 * r + 1]
        cp = cp_next

    gmax_vec = jnp.zeros((16,), jnp.float32)
    gidx_vec = jnp.zeros((16,), jnp.int32)
    for r in range(8):
        gmax = jnp.max(maxv[r], axis=0)
        ei = maxj[r] * 16 + lanes + col0
        cand = jnp.where(maxv[r] == gmax, ei, _BIG)
        gidx = jnp.min(cand, axis=0)
        gmax_vec = jnp.where(lanes == r, gmax, gmax_vec)
        gidx_vec = jnp.where(lanes == r, gidx, gidx_vec)

    res_v[pl.ds(0, 16)] = gmax_vec
    tmp_v[pl.ds(0, 16)] = gidx_vec
    pltpu.sync_copy(res_v, maxes_hbm.at[pl.ds(wid * 16, 16)])
    pltpu.sync_copy(tmp_v, idxs_hbm.at[pl.ds(wid * 16, 16)])


_sc_argmax = pl.kernel(
    _sc_argmax_body,
    out_type=(
        jax.ShapeDtypeStruct((512,), jnp.float32),
        jax.ShapeDtypeStruct((512,), jnp.int32),
    ),
    mesh=plsc.VectorSubcoreMesh(core_axis_name="c", subcore_axis_name="s"),
    scratch_types=[
        pltpu.VMEM((2, 8, _CHUNK), jnp.float32),
        pltpu.VMEM((16,), jnp.float32),
        pltpu.VMEM((16,), jnp.int32),
        pltpu.SemaphoreType.DMA,
        pltpu.SemaphoreType.DMA,
    ],
    compiler_params=pltpu.CompilerParams(needs_layout_passes=False),
)


def _onehot_body(max2_ref, idx2_ref, o_ref):
    m = max2_ref[...]
    i = idx2_ref[...]
    take_r = m[:, 1:2] > m[:, 0:1]
    idx = jnp.where(take_r, i[:, 1:2], i[:, 0:1])
    iota = lax.broadcasted_iota(jnp.int32, (32, _N), 1)
    o_ref[...] = (iota == idx).astype(jnp.int8)


def _onehot_tc(max2, idx2):
    return pl.pallas_call(
        _onehot_body,
        grid=(_R // 32,),
        in_specs=[
            pl.BlockSpec((32, 2), lambda i: (i, 0)),
            pl.BlockSpec((32, 2), lambda i: (i, 0)),
        ],
        out_specs=pl.BlockSpec((32, _N), lambda i: (i, 0)),
        out_shape=jax.ShapeDtypeStruct((_R, _N), jnp.int8),
    )(max2, idx2)


def kernel(probs):
    maxes, idxs = _sc_argmax(probs)
    max2 = maxes.reshape(16, 2, 16)[:, :, :8].transpose(0, 2, 1).reshape(_R, 2)
    idx2 = idxs.reshape(16, 2, 16)[:, :, :8].transpose(0, 2, 1).reshape(_R, 2)
    oh8 = _onehot_tc(max2, idx2)
    return oh8.astype(jnp.bool_)
